# baseline (speedup 1.0000x reference)
import functools

import jax
import jax.numpy as jnp
from jax import lax
from jax.experimental import pallas as pl
from jax.experimental.pallas import tpu as pltpu
from jax.experimental.pallas import tpu_sc as plsc

NUM_CORES = 2
NUM_SUBCORES = 16
NW = NUM_CORES * NUM_SUBCORES
IDX_LANES = 128


def _sc_gather(table, idx3d, n_rows, d):
    per_w = n_rows // NW
    idx_rows = per_w // IDX_LANES
    group = next(g for g in (5, 4, 3, 2, 1) if idx_rows % g == 0)
    rows_per_group = IDX_LANES * group
    groups = per_w // rows_per_group
    mesh = plsc.VectorSubcoreMesh(core_axis_name="c", subcore_axis_name="s")

    @functools.partial(
        pl.kernel,
        mesh=mesh,
        out_type=jax.ShapeDtypeStruct((n_rows, d), table.dtype),
        scratch_types=[
            pltpu.VMEM((idx_rows, IDX_LANES), jnp.int32),
            pltpu.VMEM((rows_per_group, d), table.dtype),
            pltpu.SemaphoreType.DMA,
        ],
    )
    def gather_kernel(table_hbm, idx_hbm, out_hbm, idx_v, rows_v, sem):
        wid = lax.axis_index("s") * NUM_CORES + lax.axis_index("c")
        row0 = wid * per_w
        pltpu.sync_copy(idx_hbm.at[wid], idx_v)

        def body(g, carry):
            copies = [
                pltpu.make_async_copy(
                    table_hbm.at[idx_v.at[g * group + b]],
                    rows_v.at[pl.ds(b * IDX_LANES, IDX_LANES)],
                    sem,
                )
                for b in range(group)
            ]
            for c in copies:
                c.start()
            for c in copies:
                c.wait()
            pltpu.sync_copy(
                rows_v, out_hbm.at[pl.ds(row0 + g * rows_per_group, rows_per_group)])
            return carry

        lax.fori_loop(0, groups, body, 0)

    return gather_kernel(table, idx3d)


def _fused(g3, enc_wt, enc_b2d, dec_w, dec_b2d, b_total, blk0, prev,
           bt=128, kc=40):
    bch, ctx, e = g3.shape
    k = ctx * e
    nb, nk = bch // bt, ctx // kc

    def body(g_ref, wt_ref, eb_ref, dw_ref, db_ref, *refs):
        out_ref, acc_ref = refs[-2], refs[-1]
        kk = pl.program_id(1)
        base = kk * kc * e
        part = lax.dot_general(
            g_ref[:, 0, :], wt_ref[pl.ds(base, e), :],
            (((1,), (0,)), ((), ())),
            precision=lax.Precision.DEFAULT,
            preferred_element_type=jnp.float32)
        for j in range(1, kc):
            part += lax.dot_general(
                g_ref[:, j, :], wt_ref[pl.ds(base + j * e, e), :],
                (((1,), (0,)), ((), ())),
                precision=lax.Precision.DEFAULT,
                preferred_element_type=jnp.float32)

        @pl.when(kk == 0)
        def _():
            acc_ref[...] = part

        @pl.when(kk > 0)
        def _():
            acc_ref[...] = acc_ref[...] + part

        @pl.when(kk == nk - 1)
        def _():
            enc = (acc_ref[...] + eb_ref[...]).astype(jnp.bfloat16)
            for j in range(ctx):
                res = lax.dot_general(
                    enc, dw_ref[j * e:(j + 1) * e, :], (((1,), (1,)), ((), ())),
                    preferred_element_type=jnp.float32)
                out_ref[:, j, :] = res + db_ref[0:1, j * e:(j + 1) * e]

    in_specs = [
        pl.BlockSpec((bt, kc, e), lambda ii, kk: (ii, kk, 0)),
        pl.BlockSpec((k, e), lambda ii, kk: (0, 0)),
        pl.BlockSpec((1, e), lambda ii, kk: (0, 0)),
        pl.BlockSpec((k, e), lambda ii, kk: (0, 0)),
        pl.BlockSpec((1, k), lambda ii, kk: (0, 0)),
    ]
    args = [g3, enc_wt, enc_b2d, dec_w, dec_b2d]
    alias = {}
    if prev is not None:
        in_specs.append(pl.BlockSpec(memory_space=pl.ANY))
        args.append(prev)
        alias = {5: 0}

    return pl.pallas_call(
        body,
        grid=(nb, nk),
        in_specs=in_specs,
        out_specs=pl.BlockSpec((bt, ctx, e), lambda ii, kk: (blk0 + ii, 0, 0)),
        out_shape=jax.ShapeDtypeStruct((b_total, ctx, e), jnp.float32),
        input_output_aliases=alias,
        scratch_shapes=[pltpu.VMEM((bt, e), jnp.float32)],
    )(*args)


def kernel(context, emb, enc_w, enc_b, dec_w, dec_b):
    b, ctx = context.shape
    _, e = emb.shape
    nch = 4
    bch = b // nch
    bt = 128
    enc_b2d = enc_b.reshape(1, e)
    dec_b2d = dec_b.reshape(1, ctx * e)
    enc_wt = enc_w.T
    dec_w_bf = dec_w.astype(jnp.bfloat16)

    out = None
    for c in range(nch):
        idx_c = context[c * bch:(c + 1) * bch]
        idx3d = idx_c.reshape(NW, bch * ctx // NW // IDX_LANES, IDX_LANES)
        gathered = _sc_gather(emb, idx3d, bch * ctx, e)
        g3 = gathered.reshape(bch, ctx, e)
        out = _fused(g3, enc_wt, enc_b2d, dec_w_bf, dec_b2d,
                     b, c * (bch // bt), out, bt=bt)
    return out


# speedup vs baseline: 1.1285x; 1.1285x over previous
import functools

import jax
import jax.numpy as jnp
from jax import lax
from jax.experimental import pallas as pl
from jax.experimental.pallas import tpu as pltpu
from jax.experimental.pallas import tpu_sc as plsc

NUM_CORES = 2
NUM_SUBCORES = 16
NW = NUM_CORES * NUM_SUBCORES
IDX_LANES = 128


def _sc_gather(table, idx3d, n_rows, d):
    per_w = n_rows // NW
    idx_rows = per_w // IDX_LANES
    group = next(g for g in (5, 4, 3, 2, 1) if idx_rows % g == 0)
    rows_per_group = IDX_LANES * group
    groups = per_w // rows_per_group
    mesh = plsc.VectorSubcoreMesh(core_axis_name="c", subcore_axis_name="s")

    @functools.partial(
        pl.kernel,
        mesh=mesh,
        out_type=jax.ShapeDtypeStruct((n_rows, d), table.dtype),
        scratch_types=[
            pltpu.VMEM((idx_rows, IDX_LANES), jnp.int32),
            pltpu.VMEM((rows_per_group, d), table.dtype),
            pltpu.SemaphoreType.DMA,
        ],
    )
    def gather_kernel(table_hbm, idx_hbm, out_hbm, idx_v, rows_v, sem):
        wid = lax.axis_index("s") * NUM_CORES + lax.axis_index("c")
        row0 = wid * per_w
        pltpu.sync_copy(idx_hbm.at[wid], idx_v)

        def body(g, carry):
            copies = [
                pltpu.make_async_copy(
                    table_hbm.at[idx_v.at[g * group + b]],
                    rows_v.at[pl.ds(b * IDX_LANES, IDX_LANES)],
                    sem,
                )
                for b in range(group)
            ]
            for c in copies:
                c.start()
            for c in copies:
                c.wait()
            pltpu.sync_copy(
                rows_v, out_hbm.at[pl.ds(row0 + g * rows_per_group, rows_per_group)])
            return carry

        lax.fori_loop(0, groups, body, 0)

    return gather_kernel(table, idx3d)


def _fused(g3, enc_wt, enc_b2d, dec_w, dec_b2d, b_total, blk0, prev,
           bt=64, kc=200):
    bch, ctx, e = g3.shape
    k = ctx * e
    nb, nk = bch // bt, ctx // kc

    def body(g_ref, wt_ref, eb_ref, dw_ref, db_ref, *refs):
        out_ref, acc_ref = refs[-2], refs[-1]
        kk = pl.program_id(1)
        part = lax.dot_general(
            g_ref[:, 0, :], wt_ref[0:e, :],
            (((1,), (0,)), ((), ())),
            precision=lax.Precision.DEFAULT,
            preferred_element_type=jnp.float32)
        for j in range(1, kc):
            part += lax.dot_general(
                g_ref[:, j, :], wt_ref[j * e:(j + 1) * e, :],
                (((1,), (0,)), ((), ())),
                precision=lax.Precision.DEFAULT,
                preferred_element_type=jnp.float32)

        @pl.when(kk == 0)
        def _():
            acc_ref[...] = part

        @pl.when(kk > 0)
        def _():
            acc_ref[...] = acc_ref[...] + part

        @pl.when(kk == nk - 1)
        def _():
            enc = (acc_ref[...] + eb_ref[...]).astype(jnp.bfloat16)
            for j in range(ctx):
                res = lax.dot_general(
                    enc, dw_ref[j * e:(j + 1) * e, :], (((1,), (1,)), ((), ())),
                    preferred_element_type=jnp.float32)
                out_ref[:, j, :] = res + db_ref[0:1, j * e:(j + 1) * e]

    in_specs = [
        pl.BlockSpec((bt, kc, e), lambda ii, kk: (ii, kk, 0)),
        pl.BlockSpec((k, e), lambda ii, kk: (0, 0)),
        pl.BlockSpec((1, e), lambda ii, kk: (0, 0)),
        pl.BlockSpec((k, e), lambda ii, kk: (0, 0)),
        pl.BlockSpec((1, k), lambda ii, kk: (0, 0)),
    ]
    args = [g3, enc_wt, enc_b2d, dec_w, dec_b2d]
    alias = {}
    if prev is not None:
        in_specs.append(pl.BlockSpec(memory_space=pl.ANY))
        args.append(prev)
        alias = {5: 0}

    return pl.pallas_call(
        body,
        grid=(nb, nk),
        in_specs=in_specs,
        out_specs=pl.BlockSpec((bt, ctx, e), lambda ii, kk: (blk0 + ii, 0, 0)),
        out_shape=jax.ShapeDtypeStruct((b_total, ctx, e), jnp.float32),
        input_output_aliases=alias,
        scratch_shapes=[pltpu.VMEM((bt, e), jnp.float32)],
    )(*args)


def kernel(context, emb, enc_w, enc_b, dec_w, dec_b):
    b, ctx = context.shape
    _, e = emb.shape
    nch = 4
    bch = b // nch
    bt = 64
    enc_b2d = enc_b.reshape(1, e)
    dec_b2d = dec_b.reshape(1, ctx * e)
    enc_wt = enc_w.T
    dec_w_bf = dec_w.astype(jnp.bfloat16)

    out = None
    for c in range(nch):
        idx_c = context[c * bch:(c + 1) * bch]
        idx3d = idx_c.reshape(NW, bch * ctx // NW // IDX_LANES, IDX_LANES)
        gathered = _sc_gather(emb, idx3d, bch * ctx, e)
        g3 = gathered.reshape(bch, ctx, e)
        out = _fused(g3, enc_wt, enc_b2d, dec_w_bf, dec_b2d,
                     b, c * (bch // bt), out, bt=bt)
    return out
